# Initial kernel scaffold; baseline (speedup 1.0000x reference)
import jax
import jax.numpy as jnp
from jax.experimental import pallas as pl


def kernel(x1, adj1, x2, adj2, W1, b1, W2, b2):
    raise NotImplementedError("write your pallas kernel here")


# speedup vs baseline: 1.0722x; 1.0722x over previous
import jax
import jax.numpy as jnp
from jax import lax
from jax.experimental import pallas as pl
from jax.experimental.pallas import tpu as pltpu

_N = 4096
_NFEAT = 128
_NHID = 16
_NCLASS = 16
_BR = 512
_NB = _N // _BR
_EPS = 1e-8


def _gcn_kernel(adj1_ref, adj2_ref, x1_ref, x2_ref, w1_ref, b1_ref, w2_ref,
                b2_ref, out_ref, xw1_ref, xw2_ref, h1a_ref, h1b_ref,
                p1_ref, p2_ref):
    i = pl.program_id(0)
    b = lax.rem(i, _NB)
    phase = i // _NB

    @pl.when(i == 0)
    def _init():
        xw1_ref[...] = jnp.dot(x1_ref[...], w1_ref[...],
                               preferred_element_type=jnp.float32)
        xw2_ref[...] = jnp.dot(x2_ref[...], w1_ref[...],
                               preferred_element_type=jnp.float32)
        p1_ref[...] = jnp.full(p1_ref.shape, -jnp.inf, jnp.float32)
        p2_ref[...] = jnp.full(p2_ref.shape, -jnp.inf, jnp.float32)

    @pl.when(phase == 0)
    def _layer1():
        h1 = jnp.dot(adj1_ref[...], xw1_ref[...],
                     preferred_element_type=jnp.float32) + b1_ref[...]
        h1a_ref[pl.ds(b * _BR, _BR), :] = jnp.maximum(h1, 0.0)
        h2 = jnp.dot(adj2_ref[...], xw2_ref[...],
                     preferred_element_type=jnp.float32) + b1_ref[...]
        h1b_ref[pl.ds(b * _BR, _BR), :] = jnp.maximum(h2, 0.0)

    @pl.when(phase == 1)
    def _layer2():
        t1 = jnp.dot(adj1_ref[...], h1a_ref[...],
                     preferred_element_type=jnp.float32)
        o1 = jnp.dot(t1, w2_ref[...],
                     preferred_element_type=jnp.float32) + b2_ref[...]
        p1_ref[...] = jnp.maximum(p1_ref[...],
                                  jnp.max(o1, axis=0, keepdims=True))
        t2 = jnp.dot(adj2_ref[...], h1b_ref[...],
                     preferred_element_type=jnp.float32)
        o2 = jnp.dot(t2, w2_ref[...],
                     preferred_element_type=jnp.float32) + b2_ref[...]
        p2_ref[...] = jnp.maximum(p2_ref[...],
                                  jnp.max(o2, axis=0, keepdims=True))

    @pl.when(i == 2 * _NB - 1)
    def _final():
        p1 = p1_ref[0, :]
        p2 = p2_ref[0, :]
        d = jnp.sum(p1 * p2)
        n1 = jnp.maximum(jnp.sqrt(jnp.sum(p1 * p1)), _EPS)
        n2 = jnp.maximum(jnp.sqrt(jnp.sum(p2 * p2)), _EPS)
        out_ref[0, 0] = jnp.abs(5.0 * d / (n1 * n2))


def _adj_spec():
    return pl.BlockSpec((_BR, _N), lambda i: (lax.rem(i, _NB), 0))


def _const_spec(shape):
    return pl.BlockSpec(shape, lambda i: tuple(0 for _ in shape))


@jax.jit
def kernel(x1, adj1, x2, adj2, W1, b1, W2, b2):
    b1r = b1.reshape(1, _NHID)
    b2r = b2.reshape(1, _NCLASS)
    out = pl.pallas_call(
        _gcn_kernel,
        grid=(2 * _NB,),
        in_specs=[
            _adj_spec(),
            _adj_spec(),
            _const_spec((_N, _NFEAT)),
            _const_spec((_N, _NFEAT)),
            _const_spec((_NFEAT, _NHID)),
            _const_spec((1, _NHID)),
            _const_spec((_NHID, _NCLASS)),
            _const_spec((1, _NCLASS)),
        ],
        out_specs=pl.BlockSpec(memory_space=pltpu.SMEM),
        out_shape=jax.ShapeDtypeStruct((1, 1), jnp.float32),
        scratch_shapes=[
            pltpu.VMEM((_N, _NHID), jnp.float32),
            pltpu.VMEM((_N, _NHID), jnp.float32),
            pltpu.VMEM((_N, _NHID), jnp.float32),
            pltpu.VMEM((_N, _NHID), jnp.float32),
            pltpu.VMEM((1, _NCLASS), jnp.float32),
            pltpu.VMEM((1, _NCLASS), jnp.float32),
        ],
        compiler_params=pltpu.CompilerParams(
            vmem_limit_bytes=100 * 1024 * 1024),
    )(adj1, adj2, x1, x2, W1, b1r, W2, b2r)
    return out
